# baseline (speedup 1.0000x reference)
import functools

import jax
import jax.numpy as jnp
from jax import lax
from jax.experimental import pallas as pl
from jax.experimental.pallas import tpu as pltpu
from jax.experimental.pallas import tpu_sc as plsc

N_TOKENS = 32768
D_MODEL = 768
N_EXP = 64
BM = 4096

_INFO = plsc.get_sparse_core_info()
NC, NS, L = _INFO.num_cores, _INFO.num_subcores, _INFO.num_lanes
NW = NC * NS
TPW = N_TOKENS // NW
GPW = TPW // L


HALF = N_TOKENS // 2


def _logits_body(x0_ref, x1_ref, w_ref, b_ref, lt0_ref, lt1_ref):
    w = w_ref[...]
    bias = b_ref[...]
    lt0_ref[...] = lax.dot_general(
        w, x0_ref[...], (((1,), (1,)), ((), ())),
        preferred_element_type=jnp.float32,
    ) + bias
    lt1_ref[...] = lax.dot_general(
        w, x1_ref[...], (((1,), (1,)), ((), ())),
        preferred_element_type=jnp.float32,
    ) + bias


def _logits_tc(x, W, b):
    nsteps = HALF // BM
    return pl.pallas_call(
        _logits_body,
        grid=(nsteps,),
        in_specs=[
            pl.BlockSpec((BM, D_MODEL), lambda i: (i, 0)),
            pl.BlockSpec((BM, D_MODEL), lambda i, n=nsteps: (i + n, 0)),
            pl.BlockSpec((N_EXP, D_MODEL), lambda i: (0, 0)),
            pl.BlockSpec((N_EXP, 1), lambda i: (0, 0)),
        ],
        out_specs=[
            pl.BlockSpec((N_EXP, BM), lambda i: (0, i)),
            pl.BlockSpec((N_EXP, BM), lambda i: (0, i)),
        ],
        out_shape=[
            jax.ShapeDtypeStruct((N_EXP, HALF), jnp.float32),
            jax.ShapeDtypeStruct((N_EXP, HALF), jnp.float32),
        ],
        compiler_params=pltpu.CompilerParams(
            dimension_semantics=("arbitrary",),
        ),
    )(x, x, W, b.reshape(N_EXP, 1))


def _router_sc(lt0_hbm, lt1_hbm, ow_hbm, oi_hbm, lv, owv, oiv):
    wid = lax.axis_index("s") * NC + lax.axis_index("c")
    base = wid * TPW

    @pl.when(wid < NW // 2)
    def _():
        pltpu.sync_copy(lt0_hbm.at[:, pl.ds(base, TPW)], lv)

    @pl.when(wid >= NW // 2)
    def _():
        pltpu.sync_copy(lt1_hbm.at[:, pl.ds(base - HALF, TPW)], lv)

    def group(g, carry):
        off = g * L
        m1 = lv[0, pl.ds(off, L)]
        i1 = jnp.zeros((L,), jnp.int32)
        m2 = jnp.full((L,), -jnp.inf, jnp.float32)
        i2 = jnp.zeros((L,), jnp.int32)
        for e in range(1, N_EXP):
            v = lv[e, pl.ds(off, L)]
            ev = jnp.full((L,), e, jnp.int32)
            gt2 = v > m2
            new1 = v > m1
            m2c = jnp.where(gt2, v, m2)
            i2c = jnp.where(gt2, ev, i2)
            m2 = jnp.where(new1, m1, m2c)
            i2 = jnp.where(new1, i1, i2c)
            m1 = jnp.where(new1, v, m1)
            i1 = jnp.where(new1, ev, i1)
        t = jnp.exp(m2 - m1)
        denom = 1.0 + t
        sl = pl.ds(off, L)
        owv[0, sl] = 1.0 / denom
        owv[1, sl] = t / denom
        oiv[0, sl] = i1
        oiv[1, sl] = i2
        return carry

    lax.fori_loop(0, GPW, group, 0)
    pltpu.sync_copy(owv, ow_hbm.at[:, pl.ds(base, TPW)])
    pltpu.sync_copy(oiv, oi_hbm.at[:, pl.ds(base, TPW)])


@jax.jit
def kernel(x, W, b):
    lt0, lt1 = _logits_tc(x, W, b)
    sc = pl.kernel(
        _router_sc,
        mesh=plsc.VectorSubcoreMesh(core_axis_name="c", subcore_axis_name="s"),
        out_type=[
            jax.ShapeDtypeStruct((2, N_TOKENS), jnp.float32),
            jax.ShapeDtypeStruct((2, N_TOKENS), jnp.int32),
        ],
        scratch_types=[
            pltpu.VMEM((N_EXP, TPW), jnp.float32),
            pltpu.VMEM((2, TPW), jnp.float32),
            pltpu.VMEM((2, TPW), jnp.int32),
        ],
    )
    ow, oi = sc(lt0, lt1)
    return ow.T, oi.T


# speedup vs baseline: 1.0247x; 1.0247x over previous
import functools

import jax
import jax.numpy as jnp
from jax import lax
from jax.experimental import pallas as pl
from jax.experimental.pallas import tpu as pltpu
from jax.experimental.pallas import tpu_sc as plsc

N_TOKENS = 32768
D_MODEL = 768
N_EXP = 64
BM = 4096

_INFO = plsc.get_sparse_core_info()
NC, NS, L = _INFO.num_cores, _INFO.num_subcores, _INFO.num_lanes
NW = NC * NS
TPW = N_TOKENS // NW
GPW = TPW // L


def _logits_body(x_ref, w_ref, b_ref, lt_ref):
    lt_ref[...] = lax.dot_general(
        w_ref[...], x_ref[...], (((1,), (1,)), ((), ())),
        preferred_element_type=jnp.float32,
    ) + b_ref[...]


def _logits_tc(x, W, b):
    return pl.pallas_call(
        _logits_body,
        grid=(N_TOKENS // BM,),
        in_specs=[
            pl.BlockSpec((BM, D_MODEL), lambda i: (i, 0)),
            pl.BlockSpec((N_EXP, D_MODEL), lambda i: (0, 0)),
            pl.BlockSpec((N_EXP, 1), lambda i: (0, 0)),
        ],
        out_specs=pl.BlockSpec((N_EXP, BM), lambda i: (0, i)),
        out_shape=jax.ShapeDtypeStruct((N_EXP, N_TOKENS), jnp.float32),
        compiler_params=pltpu.CompilerParams(
            dimension_semantics=("arbitrary",),
        ),
    )(x, W, b.reshape(N_EXP, 1))


def _router_sc(lt_hbm, ow_hbm, oi_hbm, lv, owv, oiv):
    wid = lax.axis_index("s") * NC + lax.axis_index("c")
    base = wid * TPW
    pltpu.sync_copy(lt_hbm.at[:, pl.ds(base, TPW)], lv)

    def group(g, carry):
        off = g * L
        m1 = lv[0, pl.ds(off, L)]
        i1 = jnp.zeros((L,), jnp.int32)
        m2 = jnp.full((L,), -jnp.inf, jnp.float32)
        i2 = jnp.zeros((L,), jnp.int32)
        for e in range(1, N_EXP):
            v = lv[e, pl.ds(off, L)]
            ev = jnp.full((L,), e, jnp.int32)
            gt2 = v > m2
            new1 = v > m1
            m2c = jnp.where(gt2, v, m2)
            i2c = jnp.where(gt2, ev, i2)
            m2 = jnp.where(new1, m1, m2c)
            i2 = jnp.where(new1, i1, i2c)
            m1 = jnp.where(new1, v, m1)
            i1 = jnp.where(new1, ev, i1)
        t = jnp.exp(m2 - m1)
        denom = 1.0 + t
        sl = pl.ds(off, L)
        owv[0, sl] = 1.0 / denom
        owv[1, sl] = t / denom
        oiv[0, sl] = i1
        oiv[1, sl] = i2
        return carry

    lax.fori_loop(0, GPW, group, 0)
    pltpu.sync_copy(owv, ow_hbm.at[:, pl.ds(base, TPW)])
    pltpu.sync_copy(oiv, oi_hbm.at[:, pl.ds(base, TPW)])


@jax.jit
def kernel(x, W, b):
    lt = _logits_tc(x, W, b)
    sc = pl.kernel(
        _router_sc,
        mesh=plsc.VectorSubcoreMesh(core_axis_name="c", subcore_axis_name="s"),
        out_type=[
            jax.ShapeDtypeStruct((2, N_TOKENS), jnp.float32),
            jax.ShapeDtypeStruct((2, N_TOKENS), jnp.int32),
        ],
        scratch_types=[
            pltpu.VMEM((N_EXP, TPW), jnp.float32),
            pltpu.VMEM((2, TPW), jnp.float32),
            pltpu.VMEM((2, TPW), jnp.int32),
        ],
    )
    ow, oi = sc(lt)
    return ow.T, oi.T


# speedup vs baseline: 1.0266x; 1.0019x over previous
import jax
import jax.numpy as jnp
from jax import lax
from jax.experimental import pallas as pl
from jax.experimental.pallas import tpu as pltpu
from jax.experimental.pallas import tpu_sc as plsc

N_TOKENS = 32768
D_MODEL = 768
N_EXP = 64
BM = 4096

_INFO = plsc.get_sparse_core_info()
NC, NS, L = _INFO.num_cores, _INFO.num_subcores, _INFO.num_lanes
NW = NC * NS
TPW = N_TOKENS // NW
GPW = TPW // L


def _logits_body(x_ref, w_ref, b_ref, lt_ref):
    lt_ref[...] = lax.dot_general(
        w_ref[...], x_ref[...], (((1,), (1,)), ((), ())),
        preferred_element_type=jnp.float32,
    ) + b_ref[...]


def _logits_tc(x, W, b):
    return pl.pallas_call(
        _logits_body,
        grid=(N_TOKENS // BM,),
        in_specs=[
            pl.BlockSpec((BM, D_MODEL), lambda i: (i, 0)),
            pl.BlockSpec((N_EXP, D_MODEL), lambda i: (0, 0)),
            pl.BlockSpec((N_EXP, 1), lambda i: (0, 0)),
        ],
        out_specs=pl.BlockSpec((N_EXP, BM), lambda i: (0, i)),
        out_shape=jax.ShapeDtypeStruct((N_EXP, N_TOKENS), jnp.float32),
        compiler_params=pltpu.CompilerParams(
            dimension_semantics=("arbitrary",),
        ),
    )(x, W, b.reshape(N_EXP, 1))


def _router_sc(lt_hbm, ow_hbm, oi_hbm, lv, owv, oiv):
    wid = lax.axis_index("s") * NC + lax.axis_index("c")
    base = wid * TPW
    pltpu.sync_copy(lt_hbm.at[:, pl.ds(base, TPW)], lv)

    def group(g, carry):
        off = g * L
        m1 = lv[0, pl.ds(off, L)]
        i1 = jnp.zeros((L,), jnp.int32)
        m2 = jnp.full((L,), -jnp.inf, jnp.float32)
        i2 = jnp.zeros((L,), jnp.int32)
        for e in range(1, N_EXP):
            v = lv[e, pl.ds(off, L)]
            ev = jnp.full((L,), e, jnp.int32)
            gt2 = v > m2
            new1 = v > m1
            m2c = jnp.where(gt2, v, m2)
            i2c = jnp.where(gt2, ev, i2)
            m2 = jnp.where(new1, m1, m2c)
            i2 = jnp.where(new1, i1, i2c)
            m1 = jnp.where(new1, v, m1)
            i1 = jnp.where(new1, ev, i1)
        t = jnp.exp(m2 - m1)
        denom = 1.0 + t
        sl = pl.ds(off, L)
        owv[0, sl] = 1.0 / denom
        owv[1, sl] = t / denom
        oiv[0, sl] = i1
        oiv[1, sl] = i2
        return carry

    lax.fori_loop(0, GPW, group, 0)
    pltpu.sync_copy(owv, ow_hbm.at[:, pl.ds(base, TPW)])
    pltpu.sync_copy(oiv, oi_hbm.at[:, pl.ds(base, TPW)])


@jax.jit
def kernel(x, W, b):
    lt = _logits_tc(x, W, b)
    sc = pl.kernel(
        _router_sc,
        mesh=plsc.VectorSubcoreMesh(core_axis_name="c", subcore_axis_name="s"),
        out_type=[
            jax.ShapeDtypeStruct((2, N_TOKENS), jnp.float32),
            jax.ShapeDtypeStruct((2, N_TOKENS), jnp.int32),
        ],
        scratch_types=[
            pltpu.VMEM((N_EXP, TPW), jnp.float32),
            pltpu.VMEM((2, TPW), jnp.float32),
            pltpu.VMEM((2, TPW), jnp.int32),
        ],
    )
    ow, oi = sc(lt)
    return ow.T, oi.T
